# baseline (speedup 1.0000x reference)
import jax
import jax.numpy as jnp
import numpy as np
from jax import lax
from jax.experimental import pallas as pl
from jax.experimental.pallas import tpu as pltpu
from jax.experimental.pallas import tpu_sc as plsc

NC = 2
NS = 16
NW = NC * NS
L = 16

NSPEC = 20000
S_PAD = 20480
SROW = S_PAD // L
RCH = SROW // 128
ORB = SROW // NS
NREAC = 200000
CH = 6256
NV = 392
CHPAD = NV * L
LAST_BASE = NREAC - CH
HEAD = (31 * CH - LAST_BASE) // L


NQ = 4
QV = NV // NQ
QE = QV * L


def _sc_body(alpha_h, beta_h, gamma_h, zeta_h, xi_h,
             spec_h, rows_h, ab_h, consts_h,
             out_h,
             ab_v, acc_v, al_v, be_v, ga_v, ze_v, xj_v,
             i1_v, i2_v, q1_v, q2_v, cv, shared,
             sem0, sem1):
    c = lax.axis_index("c")
    s = lax.axis_index("s")
    wid = s * NC + c
    base = lax.min(wid * CH, LAST_BASE)

    cps = [
        pltpu.async_copy(h.at[pl.ds(hoff + base, CH)],
                         v.at[pl.ds(0, CH)], sem0)
        for h, hoff, v in (
            (alpha_h, 0, al_v), (beta_h, 0, be_v), (gamma_h, 0, ga_v),
            (zeta_h, 0, ze_v), (xi_h, 0, xj_v),
            (spec_h, 0, i1_v), (spec_h, NREAC, i2_v),
            (rows_h, 2 * NREAC, q1_v), (rows_h, 3 * NREAC, q2_v),
        )
    ] + [
        pltpu.async_copy(ab_h, ab_v, sem0),
        pltpu.async_copy(consts_h, cv, sem0),
    ]

    zf = jnp.zeros((L,), jnp.float32)
    zi = jnp.zeros((L,), jnp.int32)

    @plsc.parallel_loop(0, SROW, unroll=8)
    def _zero_acc(j):
        acc_v[j] = zf

    for cp in cps:
        cp.wait()

    @pl.when(wid == NW - 1)
    def _kill_overlap():
        for j in range(HEAD):
            al_v[pl.ds(j * L, L)] = zf
            ze_v[pl.ds(j * L, L)] = zf
            xj_v[pl.ds(j * L, L)] = zf

    c1 = cv[0]
    c2 = cv[1]
    crv = cv[2]
    fuv = cv[3]

    def make_body(j):
        o = j * L
        a = al_v[pl.ds(o, L)]
        b = be_v[pl.ds(o, L)]
        g = ga_v[pl.ds(o, L)]
        z = ze_v[pl.ds(o, L)]
        x = xj_v[pl.ds(o, L)]
        i1 = i1_v[pl.ds(o, L)]
        i2 = i2_v[pl.ds(o, L)]
        q1 = q1_v[pl.ds(o, L)]
        q2 = q2_v[pl.ds(o, L)]
        rate = a * jnp.exp(b * c1 - g * c2) + z * crv + x * fuv
        ab1 = plsc.load_gather(ab_v, [i1])
        ab2 = plsc.load_gather(ab_v, [i2])
        rate = rate * ab1 * ab2
        neg = -rate
        plsc.addupdate_scatter(acc_v, [q1 >> 4, q1 & 15], rate)
        plsc.addupdate_scatter(acc_v, [q2 >> 4, q2 & 15], rate)
        plsc.addupdate_scatter(acc_v, [i1 >> 4, i1 & 15], neg)
        plsc.addupdate_scatter(acc_v, [i2 >> 4, i2 & 15], neg)

    for ref in (al_v, be_v, ga_v, ze_v, xj_v):
        ref[pl.ds(CH, L)] = zf
    for ref in (i1_v, i2_v, q1_v, q2_v):
        ref[pl.ds(CH, L)] = zi

    plsc.parallel_loop(0, NV, unroll=4)(make_body)

    plsc.subcore_barrier()
    pltpu.sync_copy(acc_v, shared.at[s])
    plsc.subcore_barrier()
    rcps = [
        pltpu.async_copy(shared.at[t, pl.ds(s * ORB, ORB)],
                         acc_v.at[pl.ds(t * ORB, ORB)], sem1)
        for t in range(NS)
    ]
    for cp in rcps:
        cp.wait()

    @plsc.parallel_loop(0, ORB, unroll=2)
    def _tree_sum(r):
        v = acc_v[r]
        for t in range(1, NS):
            v = v + acc_v[t * ORB + r]
        acc_v[r] = v

    pltpu.sync_copy(acc_v.at[pl.ds(0, ORB)],
                    out_h.at[c].at[pl.ds(s * ORB, ORB)])


def _combine_body(x_ref, o_ref):
    o_ref[...] = x_ref[0] + x_ref[1]


def kernel(time, abundances, temperature, cr_rate, fuv_rate, alpha, beta,
           gamma, zeta, xi, inc_vals, pair_reac, pair_species, inc_rows,
           inc_cols):
    f32 = jnp.float32
    t = temperature.astype(f32)
    c4 = jnp.stack([jnp.log(t / 300.0), 1.0 / t,
                    cr_rate.astype(f32), fuv_rate.astype(f32)])
    consts = jnp.broadcast_to(c4[:, None], (4, L))

    mesh = plsc.VectorSubcoreMesh(core_axis_name="c", subcore_axis_name="s")
    sc = pl.kernel(
        _sc_body,
        out_type=jax.ShapeDtypeStruct((NC, SROW, L), f32),
        mesh=mesh,
        compiler_params=pltpu.CompilerParams(
            needs_layout_passes=False, use_tc_tiling_on_sc=False),
        scratch_types=[
            pltpu.VMEM((NSPEC,), f32),
            pltpu.VMEM((SROW, L), f32),
            pltpu.VMEM((CHPAD,), f32),
            pltpu.VMEM((CHPAD,), f32),
            pltpu.VMEM((CHPAD,), f32),
            pltpu.VMEM((CHPAD,), f32),
            pltpu.VMEM((CHPAD,), f32),
            pltpu.VMEM((CHPAD,), jnp.int32),
            pltpu.VMEM((CHPAD,), jnp.int32),
            pltpu.VMEM((CHPAD,), jnp.int32),
            pltpu.VMEM((CHPAD,), jnp.int32),
            pltpu.VMEM((4, L), f32),
            pltpu.VMEM_SHARED((NS, SROW, L), f32),
            pltpu.SemaphoreType.DMA,
            pltpu.SemaphoreType.DMA,
        ],
    )
    partials = sc(alpha, beta, gamma, zeta, xi,
                  pair_species, inc_rows, abundances, consts)
    out_pad = pl.pallas_call(
        _combine_body,
        out_shape=jax.ShapeDtypeStruct((160, 128), f32),
    )(partials.reshape(NC, 160, 128))
    return out_pad.reshape(S_PAD)[:NSPEC]


# speedup vs baseline: 1.1097x; 1.1097x over previous
import jax
import jax.numpy as jnp
import numpy as np
from jax import lax
from jax.experimental import pallas as pl
from jax.experimental.pallas import tpu as pltpu
from jax.experimental.pallas import tpu_sc as plsc

NC = 2
NS = 16
NW = NC * NS
L = 16

NSPEC = 20000
S_PAD = 20480
SROW = S_PAD // L
RCH = SROW // 128
ORB = SROW // NS
NREAC = 200000
CH = 6256
NV = 392
CHPAD = NV * L
LAST_BASE = NREAC - CH
HEAD = (31 * CH - LAST_BASE) // L


NQ = 4
QV = NV // NQ
QE = QV * L


def _sc_body(alpha_h, beta_h, gamma_h, zeta_h, xi_h,
             spec_h, rows_h, ab_h, consts_h,
             out_h,
             ab_v, acc_v, al_v, be_v, ga_v, ze_v, xj_v,
             i1_v, i2_v, q1_v, q2_v, iot_v, cv, shared, ab_s,
             sem0, sem1):
    c = lax.axis_index("c")
    s = lax.axis_index("s")
    wid = s * NC + c
    base = lax.min(wid * CH, LAST_BASE)

    cps = [
        pltpu.async_copy(h.at[pl.ds(hoff + base, CH)],
                         v.at[pl.ds(0, CH)], sem0)
        for h, hoff, v in (
            (alpha_h, 0, al_v), (beta_h, 0, be_v), (gamma_h, 0, ga_v),
            (zeta_h, 0, ze_v), (xi_h, 0, xj_v),
            (spec_h, 0, i1_v), (spec_h, NREAC, i2_v),
            (rows_h, 2 * NREAC, q1_v), (rows_h, 3 * NREAC, q2_v),
        )
    ] + [
        pltpu.async_copy(consts_h, cv, sem0),
    ]

    @pl.when(s == 0)
    def _stage_ab():
        pltpu.sync_copy(ab_h, ab_s)
    plsc.subcore_barrier()
    abcp = pltpu.async_copy(ab_s, ab_v, sem1)

    zf = jnp.zeros((L,), jnp.float32)
    zi = jnp.zeros((L,), jnp.int32)

    @plsc.parallel_loop(0, SROW, unroll=8)
    def _zero_acc(j):
        acc_v[j] = zf

    ii = lax.broadcasted_iota(jnp.int32, (L,), 0)
    for j in range(RCH):
        for k in range(128 // L):
            iot_v[j, pl.ds(k * L, L)] = ii + (j * 128 + k * L)

    pltpu.sync_copy(acc_v.at[pl.ds(s * ORB, ORB)],
                    shared.at[pl.ds(s * ORB, ORB)])
    plsc.subcore_barrier()

    for cp in cps:
        cp.wait()
    abcp.wait()

    @pl.when(wid == NW - 1)
    def _kill_overlap():
        for j in range(HEAD):
            al_v[pl.ds(j * L, L)] = zf
            ze_v[pl.ds(j * L, L)] = zf
            xj_v[pl.ds(j * L, L)] = zf

    c1 = cv[0]
    c2 = cv[1]
    crv = cv[2]
    fuv = cv[3]

    def make_body(j):
        o = j * L
        a = al_v[pl.ds(o, L)]
        b = be_v[pl.ds(o, L)]
        g = ga_v[pl.ds(o, L)]
        z = ze_v[pl.ds(o, L)]
        x = xj_v[pl.ds(o, L)]
        i1 = i1_v[pl.ds(o, L)]
        i2 = i2_v[pl.ds(o, L)]
        q1 = q1_v[pl.ds(o, L)]
        q2 = q2_v[pl.ds(o, L)]
        rate = a * jnp.exp(b * c1 - g * c2) + z * crv + x * fuv
        ab1 = plsc.load_gather(ab_v, [i1])
        ab2 = plsc.load_gather(ab_v, [i2])
        rate = rate * ab1 * ab2
        neg = -rate
        plsc.addupdate_scatter(acc_v, [q1 >> 4, q1 & 15], rate)
        plsc.addupdate_scatter(acc_v, [q2 >> 4, q2 & 15], rate)
        plsc.addupdate_scatter(acc_v, [i1 >> 4, i1 & 15], neg)
        plsc.addupdate_scatter(acc_v, [i2 >> 4, i2 & 15], neg)

    for ref in (al_v, be_v, ga_v, ze_v, xj_v):
        ref[pl.ds(CH, L)] = zf
    for ref in (i1_v, i2_v, q1_v, q2_v):
        ref[pl.ds(CH, L)] = zi

    plsc.parallel_loop(0, NV, unroll=4)(make_body)

    plsc.subcore_barrier()
    rcps = [
        pltpu.async_copy(acc_v.at[pl.ds(j * 128, 128)],
                         shared.at[iot_v.at[j]], sem1, add=True)
        for j in range(RCH)
    ]
    for cp in rcps:
        cp.wait()
    plsc.subcore_barrier()
    pltpu.sync_copy(shared.at[pl.ds(s * ORB, ORB)],
                    out_h.at[c].at[pl.ds(s * ORB, ORB)])


def _combine_body(x_ref, o_ref):
    o_ref[...] = x_ref[0] + x_ref[1]


def kernel(time, abundances, temperature, cr_rate, fuv_rate, alpha, beta,
           gamma, zeta, xi, inc_vals, pair_reac, pair_species, inc_rows,
           inc_cols):
    f32 = jnp.float32
    t = temperature.astype(f32)
    c4 = jnp.stack([jnp.log(t / 300.0), 1.0 / t,
                    cr_rate.astype(f32), fuv_rate.astype(f32)])
    consts = jnp.broadcast_to(c4[:, None], (4, L))

    mesh = plsc.VectorSubcoreMesh(core_axis_name="c", subcore_axis_name="s")
    sc = pl.kernel(
        _sc_body,
        out_type=jax.ShapeDtypeStruct((NC, SROW, L), f32),
        mesh=mesh,
        compiler_params=pltpu.CompilerParams(
            needs_layout_passes=False, use_tc_tiling_on_sc=False),
        scratch_types=[
            pltpu.VMEM((NSPEC,), f32),
            pltpu.VMEM((SROW, L), f32),
            pltpu.VMEM((CHPAD,), f32),
            pltpu.VMEM((CHPAD,), f32),
            pltpu.VMEM((CHPAD,), f32),
            pltpu.VMEM((CHPAD,), f32),
            pltpu.VMEM((CHPAD,), f32),
            pltpu.VMEM((CHPAD,), jnp.int32),
            pltpu.VMEM((CHPAD,), jnp.int32),
            pltpu.VMEM((CHPAD,), jnp.int32),
            pltpu.VMEM((CHPAD,), jnp.int32),
            pltpu.VMEM((RCH, 128), jnp.int32),
            pltpu.VMEM((4, L), f32),
            pltpu.VMEM_SHARED((SROW, L), f32),
            pltpu.VMEM_SHARED((NSPEC,), f32),
            pltpu.SemaphoreType.DMA,
            pltpu.SemaphoreType.DMA,
        ],
    )
    partials = sc(alpha, beta, gamma, zeta, xi,
                  pair_species, inc_rows, abundances, consts)
    out_pad = pl.pallas_call(
        _combine_body,
        out_shape=jax.ShapeDtypeStruct((160, 128), f32),
    )(partials.reshape(NC, 160, 128))
    return out_pad.reshape(S_PAD)[:NSPEC]
